# baseline (speedup 1.0000x reference)
import functools

import jax
import jax.numpy as jnp
import numpy as np
from jax import lax
from jax.experimental import pallas as pl
from jax.experimental.pallas import tpu as pltpu
from jax.experimental.pallas import tpu_sc as plsc

_B = 128
_N = 512
_L = 16
_NC = 2
_NS = 16
_NW = _NC * _NS
_RPW = _B // _NW
_CHUNKS = _N // _L

_mesh = plsc.VectorSubcoreMesh(
    core_axis_name="c", subcore_axis_name="s", num_cores=_NC, num_subcores=_NS
)


@functools.partial(
    pl.kernel,
    out_type=jax.ShapeDtypeStruct((_B, 2, _N), jnp.float32),
    mesh=_mesh,
    scratch_types=[
        pltpu.VMEM((_RPW, _N), jnp.float32),
        pltpu.VMEM((_RPW, 2, _N), jnp.float32),
        pltpu.SemaphoreType.DMA,
        pltpu.SemaphoreType.DMA,
        pltpu.SemaphoreType.DMA,
        pltpu.SemaphoreType.DMA,
    ],
    compiler_params=pltpu.CompilerParams(needs_layout_passes=False),
)
def _compact(x_hbm, idx_hbm, out_hbm, rows_v, out_v, sem_r, sem_v, sem_i, sem_o):
    wid = lax.axis_index("s") * _NC + lax.axis_index("c")
    base = wid * _RPW
    cp_rows = pltpu.make_async_copy(x_hbm.at[pl.ds(base, _RPW)], rows_v, sem_r)
    cp_vals = pltpu.make_async_copy(
        x_hbm.at[pl.ds(base, _RPW)], out_v.at[:, 1, :], sem_v
    )
    cp_idx = pltpu.make_async_copy(idx_hbm, out_v.at[:, 0, :], sem_i)
    cp_rows.start()
    cp_vals.start()
    cp_idx.start()
    cp_rows.wait()

    zf = jnp.zeros((_L,), jnp.float32)

    def detect(r):
        def det_body(c, anyz):
            return anyz | (rows_v[r, pl.ds(c * _L, _L)] == 0.0)

        anyz = lax.fori_loop(
            0, _CHUNKS, det_body, jnp.zeros((_L,), jnp.bool_), unroll=8
        )
        return jnp.any(anyz)

    def general(r):
        r_splat = jnp.full((_L,), r, jnp.int32)
        plane0 = jnp.zeros((_L,), jnp.int32)
        plane1 = jnp.ones((_L,), jnp.int32)
        iota_f = lax.iota(jnp.int32, _L).astype(jnp.float32)

        def zero_body(c, _):
            out_v[r, 0, pl.ds(c * _L, _L)] = zf
            out_v[r, 1, pl.ds(c * _L, _L)] = zf
            return 0

        lax.fori_loop(0, _CHUNKS, zero_body, 0, unroll=4)

        def pack_body(c, n_off):
            sl = pl.ds(c * _L, _L)
            v = rows_v[r, sl]
            m = v != 0.0
            dest = plsc.cumsum(m.astype(jnp.int32)) + n_off
            idx_f = iota_f + (c * _L).astype(jnp.float32)
            plsc.store_scatter(out_v, [r_splat, plane0, dest], idx_f, mask=m)
            plsc.store_scatter(out_v, [r_splat, plane1, dest], v, mask=m)
            return n_off + plsc.all_reduce_population_count(m)

        lax.fori_loop(0, _CHUNKS, pack_body, jnp.full((_L,), -1, jnp.int32))

    haszero0 = detect(0)
    cp_vals.wait()
    cp_idx.wait()

    def finish_row(r, haszero):
        @pl.when(haszero)
        def _():
            general(r)

        pltpu.make_async_copy(
            out_v.at[r], out_hbm.at[base + r], sem_o
        ).start()

    finish_row(0, haszero0)

    def row_body(r, _):
        finish_row(r, detect(r))
        return 0

    lax.fori_loop(1, _RPW, row_body, 0)

    for r in range(_RPW):
        pltpu.make_async_copy(out_v.at[r], out_hbm.at[base + r], sem_o).wait()


_IDX_CONST = np.ascontiguousarray(
    np.broadcast_to(np.arange(_N, dtype=np.float32), (_RPW, _N))
)


def kernel(mlm_logits):
    return _compact(mlm_logits, jnp.asarray(_IDX_CONST))


# speedup vs baseline: 1.0843x; 1.0843x over previous
import functools

import jax
import jax.numpy as jnp
from jax import lax
from jax.experimental import pallas as pl
from jax.experimental.pallas import tpu as pltpu
from jax.experimental.pallas import tpu_sc as plsc

_B = 128
_N = 512
_L = 16
_NC = 2
_NS = 16
_NW = _NC * _NS
_RPW = _B // _NW
_CHUNKS = _N // _L

_mesh = plsc.VectorSubcoreMesh(
    core_axis_name="c", subcore_axis_name="s", num_cores=_NC, num_subcores=_NS
)


@functools.partial(
    pl.kernel,
    out_type=jax.ShapeDtypeStruct((_B, 2, _N), jnp.float32),
    mesh=_mesh,
    scratch_types=[
        pltpu.VMEM((_RPW, _N), jnp.float32),
        pltpu.VMEM((_RPW, 2, _N), jnp.float32),
        pltpu.VMEM((2, _N), jnp.float32),
        pltpu.SemaphoreType.DMA,
        pltpu.SemaphoreType.DMA,
        pltpu.SemaphoreType.DMA,
    ],
    compiler_params=pltpu.CompilerParams(needs_layout_passes=False),
)
def _compact(x_hbm, out_hbm, rows_v, out_v, fix_v, sem_r, sem_v, sem_o):
    wid = lax.axis_index("s") * _NC + lax.axis_index("c")
    base = wid * _RPW
    cp_rows = pltpu.make_async_copy(x_hbm.at[pl.ds(base, _RPW)], rows_v, sem_r)
    cp_vals = pltpu.make_async_copy(
        x_hbm.at[pl.ds(base, _RPW)], out_v.at[:, 1, :], sem_v
    )
    cp_rows.start()
    cp_vals.start()

    iota_f = lax.iota(jnp.int32, _L).astype(jnp.float32)

    def iota_body(c, _):
        sl = pl.ds(c * _L, _L)
        idxf = iota_f + (c * _L).astype(jnp.float32)
        out_v[0, 0, sl] = idxf
        out_v[1, 0, sl] = idxf
        out_v[2, 0, sl] = idxf
        out_v[3, 0, sl] = idxf
        return 0

    lax.fori_loop(0, _CHUNKS, iota_body, 0, unroll=4)

    cp_vals.wait()
    cp_out = pltpu.make_async_copy(out_v, out_hbm.at[pl.ds(base, _RPW)], sem_o)
    cp_out.start()

    cp_rows.wait()

    def det_all_body(c, anyz):
        sl = pl.ds(c * _L, _L)
        z01 = (rows_v[0, sl] == 0.0) | (rows_v[1, sl] == 0.0)
        z23 = (rows_v[2, sl] == 0.0) | (rows_v[3, sl] == 0.0)
        return anyz | z01 | z23

    anyz = lax.fori_loop(
        0, _CHUNKS, det_all_body, jnp.zeros((_L,), jnp.bool_), unroll=4
    )

    @pl.when(jnp.any(anyz))
    def _slow():
        zf = jnp.zeros((_L,), jnp.float32)
        plane0 = jnp.zeros((_L,), jnp.int32)
        plane1 = jnp.ones((_L,), jnp.int32)
        cp_out.wait()

        def row_body(r, _):
            def det_body(c, rz):
                return rz | (rows_v[r, pl.ds(c * _L, _L)] == 0.0)

            rowz = lax.fori_loop(
                0, _CHUNKS, det_body, jnp.zeros((_L,), jnp.bool_), unroll=4
            )

            @pl.when(jnp.any(rowz))
            def _fix():
                def zero_body(c, _):
                    sl = pl.ds(c * _L, _L)
                    fix_v[0, sl] = zf
                    fix_v[1, sl] = zf
                    return 0

                lax.fori_loop(0, _CHUNKS, zero_body, 0, unroll=4)

                def pack_body(c, n_off):
                    sl = pl.ds(c * _L, _L)
                    v = rows_v[r, sl]
                    m = v != 0.0
                    dest = plsc.cumsum(m.astype(jnp.int32)) + n_off
                    idx_f = iota_f + (c * _L).astype(jnp.float32)
                    plsc.store_scatter(fix_v, [plane0, dest], idx_f, mask=m)
                    plsc.store_scatter(fix_v, [plane1, dest], v, mask=m)
                    return n_off + plsc.all_reduce_population_count(m)

                lax.fori_loop(
                    0, _CHUNKS, pack_body, jnp.full((_L,), -1, jnp.int32)
                )
                pltpu.sync_copy(fix_v, out_hbm.at[base + r])

            return 0

        lax.fori_loop(0, _RPW, row_body, 0)

    @pl.when(jnp.logical_not(jnp.any(anyz)))
    def _fast():
        cp_out.wait()


def kernel(mlm_logits):
    return _compact(mlm_logits)
